# baseline (speedup 1.0000x reference)
import jax
import jax.numpy as jnp
from jax.experimental import pallas as pl
from jax.experimental.pallas import tpu as pltpu

ALPHA = 0.7
TAU_WORD = 0.1
TOK = 128
SUB = 8
TILES = TOK // SUB


def _sim_copy(sim_hbm, sim_buf, sem, tgt_ref, slot, step, k):
    row = tgt_ref[step * TOK + k]
    return pltpu.make_async_copy(
        sim_hbm.at[pl.ds(row, 1), :],
        sim_buf.at[slot, pl.ds(k, 1), :],
        sem.at[slot, k // SUB],
    )


def _wsc_kernel(
    tgt_ref, in_ref, mask_ref, tgt2_ref, sim_hbm,
    out_ref, sim_buf, pr_acc, ss_acc, smem_acc, sem,
):
    i = pl.program_id(0)
    n = pl.num_programs(0)
    v = in_ref.shape[-1]
    slot = jax.lax.rem(i, 2)
    nxt = jax.lax.rem(i + 1, 2)

    @pl.when(i == 0)
    def _prologue():
        smem_acc[0] = 0.0
        smem_acc[1] = 0.0
        pr_acc[...] = jnp.zeros_like(pr_acc)
        ss_acc[...] = jnp.zeros_like(ss_acc)
        for k in range(TOK):
            _sim_copy(sim_hbm, sim_buf, sem, tgt_ref, 0, 0, k).start()

    col_iota = jax.lax.broadcasted_iota(jnp.int32, (SUB, v), 1)
    ml_part = jnp.zeros((SUB, 1), jnp.float32)
    for j in range(TILES):
        @pl.when(i + 1 < n)
        def _prefetch():
            for k in range(j * SUB, (j + 1) * SUB):
                _sim_copy(sim_hbm, sim_buf, sem, tgt_ref, nxt, i + 1, k).start()

        for k in range(j * SUB, (j + 1) * SUB):
            _sim_copy(sim_hbm, sim_buf, sem, tgt_ref, slot, i, k).wait()

        sim_t = sim_buf[slot, pl.ds(j * SUB, SUB), :]
        in_t = in_ref[pl.ds(j * SUB, SUB), :]
        m_t = mask_ref[pl.ds(j * SUB, SUB), :]
        tgt_t = tgt2_ref[pl.ds(j * SUB, SUB), :]
        smooth = jnp.exp((sim_t - 1.0) * (1.0 / TAU_WORD))
        t = smooth * m_t
        ss_acc[...] += t
        pr_acc[...] += in_t * t
        hit = col_iota == tgt_t
        ml_part += jnp.sum(jnp.where(hit, in_t, 0.0), axis=1, keepdims=True) * m_t
    smem_acc[0] += jnp.sum(mask_ref[...])
    smem_acc[1] += jnp.sum(ml_part)

    @pl.when(i == n - 1)
    def _fin():
        ml = -smem_acc[1] / smem_acc[0]
        smooth_loss = -jnp.sum(pr_acc[...]) / jnp.sum(ss_acc[...])
        out_ref[0] = ml
        out_ref[1] = ALPHA * smooth_loss + (1.0 - ALPHA) * ml


@jax.jit
def _run(flat_in, flat_t, mask2, tgt2, Sim_Matrix):
    n, v = flat_in.shape
    grid_spec = pltpu.PrefetchScalarGridSpec(
        num_scalar_prefetch=1,
        grid=(n // TOK,),
        in_specs=[
            pl.BlockSpec((TOK, v), lambda i, tgt: (i, 0)),
            pl.BlockSpec((TOK, 1), lambda i, tgt: (i, 0)),
            pl.BlockSpec((TOK, 1), lambda i, tgt: (i, 0)),
            pl.BlockSpec(memory_space=pltpu.HBM),
        ],
        out_specs=pl.BlockSpec(memory_space=pltpu.SMEM),
        scratch_shapes=[
            pltpu.VMEM((2, TOK, v), jnp.float32),
            pltpu.VMEM((SUB, v), jnp.float32),
            pltpu.VMEM((SUB, v), jnp.float32),
            pltpu.SMEM((2,), jnp.float32),
            pltpu.SemaphoreType.DMA((2, TILES)),
        ],
    )
    out = pl.pallas_call(
        _wsc_kernel,
        grid_spec=grid_spec,
        out_shape=jax.ShapeDtypeStruct((2,), jnp.float32),
    )(flat_t, flat_in, mask2, tgt2, Sim_Matrix)
    return out[0], out[1]


def kernel(input, target, mask, Sim_Matrix):
    b, t, v = input.shape
    flat_in = input.reshape(b * t, v)
    flat_t = target[:, :t].reshape(-1)
    mask2 = mask[:, :t].reshape(-1, 1)
    tgt2 = flat_t.reshape(-1, 1)
    return _run(flat_in, flat_t, mask2, tgt2, Sim_Matrix)


# speedup vs baseline: 1.2500x; 1.2500x over previous
import jax
import jax.numpy as jnp
from jax.experimental import pallas as pl
from jax.experimental.pallas import tpu as pltpu

ALPHA = 0.7
TAU_WORD = 0.1
SUB = 8


def _chunks(t):
    out = []
    off = 0
    while off < t:
        sz = min(SUB, t - off)
        out.append((off, sz))
        off += sz
    return out


def _sim_copy(sim_hbm, sim_buf, sem, tgt_ref, t, slot, step, k):
    row = tgt_ref[step * t + k]
    return pltpu.make_async_copy(
        sim_hbm.at[pl.ds(row, 1), :],
        sim_buf.at[slot, pl.ds(k, 1), :],
        sem.at[slot, k // SUB],
    )


def _wsc_kernel(
    tgt_ref, in_ref, mask_ref, tgt2_ref, sim_hbm,
    out_ref, sim_buf, pr_acc, ss_acc, smem_acc, sem,
):
    i = pl.program_id(0)
    n = pl.num_programs(0)
    _, t, v = in_ref.shape
    slot = jax.lax.rem(i, 2)
    nxt = jax.lax.rem(i + 1, 2)

    @pl.when(i == 0)
    def _prologue():
        smem_acc[0] = 0.0
        smem_acc[1] = 0.0
        pr_acc[...] = jnp.zeros_like(pr_acc)
        ss_acc[...] = jnp.zeros_like(ss_acc)
        for k in range(t):
            _sim_copy(sim_hbm, sim_buf, sem, tgt_ref, t, 0, 0, k).start()

    for off, sz in _chunks(t):
        @pl.when(i + 1 < n)
        def _prefetch():
            for k in range(off, off + sz):
                _sim_copy(sim_hbm, sim_buf, sem, tgt_ref, t, nxt, i + 1, k).start()

        for k in range(off, off + sz):
            _sim_copy(sim_hbm, sim_buf, sem, tgt_ref, t, slot, i, k).wait()

        sim_t = sim_buf[slot, pl.ds(off, sz), :]
        in_t = in_ref[0, pl.ds(off, sz), :]
        m_t = mask_ref[0, pl.ds(off, sz), :]
        tgt_t = tgt2_ref[0, pl.ds(off, sz), :]
        smooth = jnp.exp((sim_t - 1.0) * (1.0 / TAU_WORD))
        tm = smooth * m_t
        ss_acc[pl.ds(0, sz), :] += tm
        pr_acc[pl.ds(0, sz), :] += in_t * tm
        col_iota = jax.lax.broadcasted_iota(jnp.int32, (sz, v), 1)
        hit = col_iota == tgt_t
        smem_acc[1] += jnp.sum(
            jnp.sum(jnp.where(hit, in_t, 0.0), axis=1, keepdims=True) * m_t
        )
    smem_acc[0] += jnp.sum(mask_ref[...])

    @pl.when(i == n - 1)
    def _fin():
        ml = -smem_acc[1] / smem_acc[0]
        smooth_loss = -jnp.sum(pr_acc[...]) / jnp.sum(ss_acc[...])
        out_ref[0] = ml
        out_ref[1] = ALPHA * smooth_loss + (1.0 - ALPHA) * ml


@jax.jit
def _run(input, flat_t, mask3, tgt3, Sim_Matrix):
    b, t, v = input.shape
    grid_spec = pltpu.PrefetchScalarGridSpec(
        num_scalar_prefetch=1,
        grid=(b,),
        in_specs=[
            pl.BlockSpec((1, t, v), lambda i, tgt: (i, 0, 0)),
            pl.BlockSpec((1, t, 1), lambda i, tgt: (i, 0, 0)),
            pl.BlockSpec((1, t, 1), lambda i, tgt: (i, 0, 0)),
            pl.BlockSpec(memory_space=pltpu.HBM),
        ],
        out_specs=pl.BlockSpec(memory_space=pltpu.SMEM),
        scratch_shapes=[
            pltpu.VMEM((2, t, v), jnp.float32),
            pltpu.VMEM((SUB, v), jnp.float32),
            pltpu.VMEM((SUB, v), jnp.float32),
            pltpu.SMEM((2,), jnp.float32),
            pltpu.SemaphoreType.DMA((2, (t + SUB - 1) // SUB)),
        ],
    )
    out = pl.pallas_call(
        _wsc_kernel,
        grid_spec=grid_spec,
        out_shape=jax.ShapeDtypeStruct((2,), jnp.float32),
    )(flat_t, input, mask3, tgt3, Sim_Matrix)
    return out[0], out[1]


def kernel(input, target, mask, Sim_Matrix):
    b, t, v = input.shape
    flat_t = target[:, :t].reshape(-1)
    mask3 = mask[:, :t].reshape(b, t, 1)
    tgt3 = target[:, :t].reshape(b, t, 1)
    return _run(input, flat_t, mask3, tgt3, Sim_Matrix)


# speedup vs baseline: 1.8381x; 1.4705x over previous
import jax
import jax.numpy as jnp
from jax.experimental import pallas as pl
from jax.experimental.pallas import tpu as pltpu

ALPHA = 0.7
TAU_WORD = 0.1
SUB = 8
CW = 1280
NSLOT = 3


def _chunks(total, width):
    out = []
    off = 0
    while off < total:
        sz = min(width, total - off)
        out.append((off, sz))
        off += sz
    return out


def _sim_copy(sim_hbm, sim_buf, sem, tgt_ref, t, slot, step, k):
    row = tgt_ref[step * t + k]
    return pltpu.make_async_copy(
        sim_hbm.at[pl.ds(row, 1), :],
        sim_buf.at[slot, pl.ds(k, 1), :],
        sem.at[slot, k // SUB],
    )


def _wsc_kernel(
    tgt_ref, in_ref, mask_ref, tgt2_ref, sim_hbm,
    out_ref, sim_buf, pr_acc, ss_acc, ml_acc, smem_acc, sem,
):
    i = pl.program_id(0)
    n = pl.num_programs(0)
    _, t, v = in_ref.shape
    slot = jax.lax.rem(i, NSLOT)
    nxt = jax.lax.rem(i + 1, NSLOT)

    @pl.when(i == 0)
    def _prologue():
        smem_acc[0] = 0.0
        pr_acc[...] = jnp.zeros_like(pr_acc)
        ss_acc[...] = jnp.zeros_like(ss_acc)
        ml_acc[...] = jnp.zeros_like(ml_acc)
        for k in range(t):
            _sim_copy(sim_hbm, sim_buf, sem, tgt_ref, t, 0, 0, k).start()
        for k in range(t):
            _sim_copy(sim_hbm, sim_buf, sem, tgt_ref, t, 1, 1, k).start()

    @pl.when(i + 2 < n)
    def _prefetch():
        nxt2 = jax.lax.rem(i + 2, NSLOT)
        for k in range(t):
            _sim_copy(sim_hbm, sim_buf, sem, tgt_ref, t, nxt2, i + 2, k).start()

    for off, sz in _chunks(t, SUB):
        for k in range(off, off + sz):
            _sim_copy(sim_hbm, sim_buf, sem, tgt_ref, t, slot, i, k).wait()

    for coff, cw in _chunks(v, CW):
        pr8 = jnp.zeros((SUB, cw), jnp.float32)
        ss8 = jnp.zeros((SUB, cw), jnp.float32)
        ml8 = jnp.zeros((SUB, cw), jnp.float32)
        for off, sz in _chunks(t, SUB):
            sim_t = sim_buf[slot, pl.ds(off, sz), pl.ds(coff, cw)]
            in_t = in_ref[0, pl.ds(off, sz), pl.ds(coff, cw)]
            m_t = mask_ref[0, pl.ds(off, sz), :]
            tgt_t = tgt2_ref[0, pl.ds(off, sz), :]
            smooth = jnp.exp((sim_t - 1.0) * (1.0 / TAU_WORD))
            tm = smooth * m_t
            hit = (
                coff + jax.lax.broadcasted_iota(jnp.int32, (sz, cw), 1)
            ) == tgt_t
            mlv = jnp.where(hit, in_t, 0.0) * m_t
            if sz == SUB:
                ss8 += tm
                pr8 += in_t * tm
                ml8 += mlv
            else:
                ss_acc[pl.ds(0, sz), pl.ds(coff, cw)] += tm
                pr_acc[pl.ds(0, sz), pl.ds(coff, cw)] += in_t * tm
                ml_acc[pl.ds(0, sz), pl.ds(coff, cw)] += mlv
        ss_acc[:, pl.ds(coff, cw)] += ss8
        pr_acc[:, pl.ds(coff, cw)] += pr8
        ml_acc[:, pl.ds(coff, cw)] += ml8
    smem_acc[0] += jnp.sum(mask_ref[...])

    @pl.when(i == n - 1)
    def _fin():
        ml = -jnp.sum(ml_acc[...]) / smem_acc[0]
        smooth_loss = -jnp.sum(pr_acc[...]) / jnp.sum(ss_acc[...])
        out_ref[0] = ml
        out_ref[1] = ALPHA * smooth_loss + (1.0 - ALPHA) * ml


@jax.jit
def _run(input, flat_t, mask3, tgt3, Sim_Matrix):
    b, t, v = input.shape
    grid_spec = pltpu.PrefetchScalarGridSpec(
        num_scalar_prefetch=1,
        grid=(b,),
        in_specs=[
            pl.BlockSpec((1, t, v), lambda i, tgt: (i, 0, 0)),
            pl.BlockSpec((1, t, 1), lambda i, tgt: (i, 0, 0)),
            pl.BlockSpec((1, t, 1), lambda i, tgt: (i, 0, 0)),
            pl.BlockSpec(memory_space=pltpu.HBM),
        ],
        out_specs=pl.BlockSpec(memory_space=pltpu.SMEM),
        scratch_shapes=[
            pltpu.VMEM((NSLOT, t, v), jnp.float32),
            pltpu.VMEM((SUB, v), jnp.float32),
            pltpu.VMEM((SUB, v), jnp.float32),
            pltpu.VMEM((SUB, v), jnp.float32),
            pltpu.SMEM((1,), jnp.float32),
            pltpu.SemaphoreType.DMA((NSLOT, (t + SUB - 1) // SUB)),
        ],
    )
    out = pl.pallas_call(
        _wsc_kernel,
        grid_spec=grid_spec,
        out_shape=jax.ShapeDtypeStruct((2,), jnp.float32),
    )(flat_t, input, mask3, tgt3, Sim_Matrix)
    return out[0], out[1]


def kernel(input, target, mask, Sim_Matrix):
    b, t, v = input.shape
    flat_t = target[:, :t].reshape(-1)
    mask3 = mask[:, :t].reshape(b, t, 1)
    tgt3 = target[:, :t].reshape(b, t, 1)
    return _run(input, flat_t, mask3, tgt3, Sim_Matrix)


# speedup vs baseline: 1.8407x; 1.0014x over previous
import jax
import jax.numpy as jnp
from jax.experimental import pallas as pl
from jax.experimental.pallas import tpu as pltpu

ALPHA = 0.7
TAU_WORD = 0.1
SUB = 8
CW = 640
NSLOT = 3


def _chunks(total, width):
    out = []
    off = 0
    while off < total:
        sz = min(width, total - off)
        out.append((off, sz))
        off += sz
    return out


def _sim_copy(sim_hbm, sim_buf, sem, tgt_ref, t, slot, step, k):
    row = tgt_ref[step * t + k]
    return pltpu.make_async_copy(
        sim_hbm.at[pl.ds(row, 1), :],
        sim_buf.at[slot, pl.ds(k, 1), :],
        sem.at[slot, k // SUB],
    )


def _wsc_kernel(
    tgt_ref, in_ref, mask_ref, tgt2_ref, sim_hbm,
    out_ref, sim_buf, pr_acc, ss_acc, ml_acc, smem_acc, sem,
):
    i = pl.program_id(0)
    n = pl.num_programs(0)
    _, t, v = in_ref.shape
    slot = jax.lax.rem(i, NSLOT)
    nxt = jax.lax.rem(i + 1, NSLOT)

    @pl.when(i == 0)
    def _prologue():
        smem_acc[0] = 0.0
        pr_acc[...] = jnp.zeros_like(pr_acc)
        ss_acc[...] = jnp.zeros_like(ss_acc)
        ml_acc[...] = jnp.zeros_like(ml_acc)
        for k in range(t):
            _sim_copy(sim_hbm, sim_buf, sem, tgt_ref, t, 0, 0, k).start()
        for k in range(t):
            _sim_copy(sim_hbm, sim_buf, sem, tgt_ref, t, 1, 1, k).start()

    @pl.when(i + 2 < n)
    def _prefetch():
        nxt2 = jax.lax.rem(i + 2, NSLOT)
        for k in range(t):
            _sim_copy(sim_hbm, sim_buf, sem, tgt_ref, t, nxt2, i + 2, k).start()

    for off, sz in _chunks(t, SUB):
        for k in range(off, off + sz):
            _sim_copy(sim_hbm, sim_buf, sem, tgt_ref, t, slot, i, k).wait()

    for coff, cw in _chunks(v, CW):
        pr8 = jnp.zeros((SUB, cw), jnp.float32)
        ss8 = jnp.zeros((SUB, cw), jnp.float32)
        ml8 = jnp.zeros((SUB, cw), jnp.float32)
        for off, sz in _chunks(t, SUB):
            sim_t = sim_buf[slot, pl.ds(off, sz), pl.ds(coff, cw)]
            in_t = in_ref[0, pl.ds(off, sz), pl.ds(coff, cw)]
            m_t = mask_ref[0, pl.ds(off, sz), :]
            tgt_t = tgt2_ref[0, pl.ds(off, sz), :]
            smooth = jnp.exp((sim_t - 1.0) * (1.0 / TAU_WORD))
            tm = smooth * m_t
            hit = (
                coff + jax.lax.broadcasted_iota(jnp.int32, (sz, cw), 1)
            ) == tgt_t
            mlv = jnp.where(hit, in_t, 0.0) * m_t
            if sz == SUB:
                ss8 += tm
                pr8 += in_t * tm
                ml8 += mlv
            else:
                ss_acc[pl.ds(0, sz), pl.ds(coff, cw)] += tm
                pr_acc[pl.ds(0, sz), pl.ds(coff, cw)] += in_t * tm
                ml_acc[pl.ds(0, sz), pl.ds(coff, cw)] += mlv
        ss_acc[:, pl.ds(coff, cw)] += ss8
        pr_acc[:, pl.ds(coff, cw)] += pr8
        ml_acc[:, pl.ds(coff, cw)] += ml8
    smem_acc[0] += jnp.sum(mask_ref[...])

    @pl.when(i == n - 1)
    def _fin():
        ml = -jnp.sum(ml_acc[...]) / smem_acc[0]
        smooth_loss = -jnp.sum(pr_acc[...]) / jnp.sum(ss_acc[...])
        out_ref[0] = ml
        out_ref[1] = ALPHA * smooth_loss + (1.0 - ALPHA) * ml


@jax.jit
def _run(input, flat_t, mask3, tgt3, Sim_Matrix):
    b, t, v = input.shape
    grid_spec = pltpu.PrefetchScalarGridSpec(
        num_scalar_prefetch=1,
        grid=(b,),
        in_specs=[
            pl.BlockSpec((1, t, v), lambda i, tgt: (i, 0, 0)),
            pl.BlockSpec((1, t, 1), lambda i, tgt: (i, 0, 0)),
            pl.BlockSpec((1, t, 1), lambda i, tgt: (i, 0, 0)),
            pl.BlockSpec(memory_space=pltpu.HBM),
        ],
        out_specs=pl.BlockSpec(memory_space=pltpu.SMEM),
        scratch_shapes=[
            pltpu.VMEM((NSLOT, t, v), jnp.float32),
            pltpu.VMEM((SUB, v), jnp.float32),
            pltpu.VMEM((SUB, v), jnp.float32),
            pltpu.VMEM((SUB, v), jnp.float32),
            pltpu.SMEM((1,), jnp.float32),
            pltpu.SemaphoreType.DMA((NSLOT, (t + SUB - 1) // SUB)),
        ],
    )
    out = pl.pallas_call(
        _wsc_kernel,
        grid_spec=grid_spec,
        out_shape=jax.ShapeDtypeStruct((2,), jnp.float32),
    )(flat_t, input, mask3, tgt3, Sim_Matrix)
    return out[0], out[1]


def kernel(input, target, mask, Sim_Matrix):
    b, t, v = input.shape
    flat_t = target[:, :t].reshape(-1)
    mask3 = mask[:, :t].reshape(b, t, 1)
    tgt3 = target[:, :t].reshape(b, t, 1)
    return _run(input, flat_t, mask3, tgt3, Sim_Matrix)
